# baseline (speedup 1.0000x reference)
import functools

import jax
import jax.numpy as jnp
from jax import lax
from jax.experimental import pallas as pl
from jax.experimental.pallas import tpu as pltpu
from jax.experimental.pallas import tpu_sc as plsc

_DIM = 32
_N_EMBED = 8192
_TOK_BLOCK = 1024
_CODE_CHUNK = 4096
_N_CHUNKS = _N_EMBED // _CODE_CHUNK

_SC_CORES = 2
_SC_SUBCORES = 16
_SC_WORKERS = _SC_CORES * _SC_SUBCORES


def _tc_body(x_ref, e_ref, et_ref, bi_ref, wi_ref, cbn_ref, diff_ref,
             acc_ref, e2_ref, n2_ref, n_tok_blocks, inv_count):
    t = pl.program_id(0)
    x = x_ref[...]
    x2 = jnp.sum(x * x, axis=1, keepdims=True)

    @pl.when(t == 0)
    def _():
        e_raw = e_ref[...]
        norm = jnp.sqrt(jnp.sum(e_raw * e_raw, axis=0, keepdims=True))
        e_n = e_raw / jnp.maximum(norm, 1e-12)
        e2_ref[...] = e_n + e_n
        n2_ref[...] = jnp.sum(e_n * e_n, axis=0, keepdims=True)

    best_val = jnp.full((_TOK_BLOCK, 1), jnp.inf, jnp.float32)
    worst_val = jnp.full((_TOK_BLOCK, 1), -jnp.inf, jnp.float32)
    best_idx = jnp.zeros((_TOK_BLOCK, 1), jnp.int32)
    worst_idx = jnp.zeros((_TOK_BLOCK, 1), jnp.int32)
    iota = jax.lax.broadcasted_iota(
        jnp.int32, (_TOK_BLOCK, _CODE_CHUNK), 1).astype(jnp.float32)

    for c in range(_N_CHUNKS):
        e2 = e2_ref[:, c * _CODE_CHUNK:(c + 1) * _CODE_CHUNK]
        n2 = n2_ref[:, c * _CODE_CHUNK:(c + 1) * _CODE_CHUNK]
        s2 = jax.lax.dot_general(x, e2, (((1,), (0,)), ((), ())),
                                 preferred_element_type=jnp.float32)
        d = (x2 - s2) + n2
        bm = jnp.min(d, axis=1, keepdims=True)
        bi = jnp.argmin(d, axis=1).astype(jnp.int32).reshape(
            _TOK_BLOCK, 1) + (c * _CODE_CHUNK)
        wm = jnp.max(d, axis=1, keepdims=True)
        wi = jnp.argmax(d, axis=1).astype(jnp.int32).reshape(
            _TOK_BLOCK, 1) + (c * _CODE_CHUNK)
        upd_b = bm < best_val
        best_idx = jnp.where(upd_b, bi, best_idx)
        best_val = jnp.where(upd_b, bm, best_val)
        upd_w = wm > worst_val
        worst_idx = jnp.where(upd_w, wi, worst_idx)
        worst_val = jnp.where(upd_w, wm, worst_val)

    bi_ref[...] = best_idx
    wi_ref[...] = worst_idx

    @pl.when(t == 0)
    def _():
        for c in range(_N_CHUNKS):
            cb_raw = et_ref[c * _CODE_CHUNK:(c + 1) * _CODE_CHUNK, :]
            rnorm = jnp.sqrt(jnp.sum(cb_raw * cb_raw, axis=1, keepdims=True))
            cb = (cb_raw / jnp.maximum(rnorm, 1e-12)) * 0.5
            cbn_ref[c * _CODE_CHUNK:(c + 1) * _CODE_CHUNK, :] = (
                jnp.concatenate(
                    [cb, jnp.zeros((_CODE_CHUNK, 128 - _DIM), jnp.float32)],
                    axis=1))

    block_sum = jnp.sum(best_val) + jnp.sum(worst_val)

    @pl.when(t == 0)
    def _():
        acc_ref[0, 0] = 0.0

    acc_ref[0, 0] += block_sum

    @pl.when(t == n_tok_blocks - 1)
    def _():
        diff_ref[...] = jnp.full((1, 1), acc_ref[0, 0] * inv_count,
                                 jnp.float32)


def _tc_stage(x2d, embed, et, n_tok_blocks, inv_count):
    n_tok = x2d.shape[0]
    return pl.pallas_call(
        functools.partial(_tc_body, n_tok_blocks=n_tok_blocks,
                          inv_count=inv_count),
        grid=(n_tok_blocks,),
        in_specs=[
            pl.BlockSpec((_TOK_BLOCK, _DIM), lambda t: (t, 0)),
            pl.BlockSpec((_DIM, _N_EMBED), lambda t: (0, 0)),
            pl.BlockSpec((_N_EMBED, _DIM), lambda t: (0, 0)),
        ],
        out_specs=[
            pl.BlockSpec((_TOK_BLOCK, 1), lambda t: (t, 0)),
            pl.BlockSpec((_TOK_BLOCK, 1), lambda t: (t, 0)),
            pl.BlockSpec((_N_EMBED, 128), lambda t: (0, 0)),
            pl.BlockSpec((1, 1), lambda t: (0, 0)),
        ],
        out_shape=[
            jax.ShapeDtypeStruct((n_tok, 1), jnp.int32),
            jax.ShapeDtypeStruct((n_tok, 1), jnp.int32),
            jax.ShapeDtypeStruct((_N_EMBED, 128), jnp.float32),
            jax.ShapeDtypeStruct((1, 1), jnp.float32),
        ],
        scratch_shapes=[
            pltpu.SMEM((1, 1), jnp.float32),
            pltpu.VMEM((_DIM, _N_EMBED), jnp.float32),
            pltpu.VMEM((1, _N_EMBED), jnp.float32),
        ],
        compiler_params=pltpu.CompilerParams(
            dimension_semantics=("arbitrary",)),
    )(x2d, embed, et)


def _make_sc_gather(n_tok):
    rows_per_w = n_tok // _SC_WORKERS
    n_sub = rows_per_w // 128
    mesh = plsc.VectorSubcoreMesh(core_axis_name="c", subcore_axis_name="s")

    @functools.partial(
        pl.kernel, mesh=mesh,
        out_type=jax.ShapeDtypeStruct((n_tok, 128), jnp.float32),
        scratch_types=[
            pltpu.VMEM((n_sub, 128), jnp.int32),
            pltpu.VMEM((n_sub, 128), jnp.int32),
            pltpu.VMEM((128, 128), jnp.float32),
            pltpu.VMEM((128, 128), jnp.float32),
            pltpu.SemaphoreType.DMA,
            pltpu.SemaphoreType.DMA,
        ],
    )
    def sc_gather(tbl_hbm, bi_hbm, wi_hbm, out_hbm, bi_v, wi_v, q_v, fq_v,
                  sem1, sem2):
        wid = lax.axis_index("s") * _SC_CORES + lax.axis_index("c")
        base = wid * rows_per_w
        pltpu.sync_copy(bi_hbm.at[wid], bi_v)
        pltpu.sync_copy(wi_hbm.at[wid], wi_v)
        for k in range(n_sub):
            cp1 = pltpu.async_copy(tbl_hbm.at[bi_v.at[k]], q_v, sem1)
            cp2 = pltpu.async_copy(tbl_hbm.at[wi_v.at[k]], fq_v, sem2)
            cp1.wait()
            cp2.wait()

            def body(r, carry):
                q_v[r, 0:16] = q_v[r, 0:16] + fq_v[r, 0:16]
                q_v[r, 16:32] = q_v[r, 16:32] + fq_v[r, 16:32]
                return carry

            lax.fori_loop(0, 128, body, 0)
            pltpu.sync_copy(q_v, out_hbm.at[pl.ds(base + k * 128, 128)])

    return sc_gather


def kernel(input, embed):
    n_tok = input.shape[0] * input.shape[1]
    n_tok_blocks = n_tok // _TOK_BLOCK
    inv_count = 1.0 / (n_tok * _DIM)
    x2d = input.reshape(n_tok, _DIM)
    et = embed.T

    bi, wi, cbn_half, diff11 = _tc_stage(x2d, embed, et, n_tok_blocks,
                                         inv_count)
    n_sub = (n_tok // _SC_WORKERS) // 128
    mix_pad = _make_sc_gather(n_tok)(
        cbn_half, bi.reshape(_SC_WORKERS, n_sub, 128),
        wi.reshape(_SC_WORKERS, n_sub, 128))

    mix = mix_pad[:, :_DIM].reshape(input.shape)
    diff = diff11[0, 0]
    return (mix, diff)


# speedup vs baseline: 1.1191x; 1.1191x over previous
import functools

import jax
import jax.numpy as jnp
from jax import lax
from jax.experimental import pallas as pl
from jax.experimental.pallas import tpu as pltpu
from jax.experimental.pallas import tpu_sc as plsc

_DIM = 32
_N_EMBED = 8192
_TOK_BLOCK = 2048
_CODE_CHUNK = 2048
_N_CHUNKS = _N_EMBED // _CODE_CHUNK

_SC_CORES = 2
_SC_SUBCORES = 16
_SC_WORKERS = _SC_CORES * _SC_SUBCORES


def _tc_body(x_ref, e_ref, et_ref, bi_ref, wi_ref, cbn_ref, diff_ref,
             acc_ref, e2_ref, n2_ref, n_tok_blocks, inv_count):
    t = pl.program_id(0)
    x = x_ref[...]
    x2 = jnp.sum(x * x, axis=1, keepdims=True)

    @pl.when(t == 0)
    def _():
        e_raw = e_ref[...]
        norm = jnp.sqrt(jnp.sum(e_raw * e_raw, axis=0, keepdims=True))
        e_n = e_raw / jnp.maximum(norm, 1e-12)
        e2_ref[...] = e_n + e_n
        n2_ref[...] = jnp.sum(e_n * e_n, axis=0, keepdims=True)

    best_val = jnp.full((_TOK_BLOCK, 1), jnp.inf, jnp.float32)
    worst_val = jnp.full((_TOK_BLOCK, 1), -jnp.inf, jnp.float32)
    best_idx = jnp.zeros((_TOK_BLOCK, 1), jnp.int32)
    worst_idx = jnp.zeros((_TOK_BLOCK, 1), jnp.int32)
    iota = jax.lax.broadcasted_iota(
        jnp.int32, (_TOK_BLOCK, _CODE_CHUNK), 1).astype(jnp.float32)

    for c in range(_N_CHUNKS):
        e2 = e2_ref[:, c * _CODE_CHUNK:(c + 1) * _CODE_CHUNK]
        n2 = n2_ref[:, c * _CODE_CHUNK:(c + 1) * _CODE_CHUNK]
        s2 = jax.lax.dot_general(x, e2, (((1,), (0,)), ((), ())),
                                 preferred_element_type=jnp.float32)
        d = (x2 - s2) + n2
        bm = jnp.min(d, axis=1, keepdims=True)
        bi = jnp.min(jnp.where(d == bm, iota, jnp.float32(_N_EMBED)),
                     axis=1, keepdims=True).astype(jnp.int32) + (
                         c * _CODE_CHUNK)
        wm = jnp.max(d, axis=1, keepdims=True)
        wi = jnp.min(jnp.where(d == wm, iota, jnp.float32(_N_EMBED)),
                     axis=1, keepdims=True).astype(jnp.int32) + (
                         c * _CODE_CHUNK)
        upd_b = bm < best_val
        best_idx = jnp.where(upd_b, bi, best_idx)
        best_val = jnp.where(upd_b, bm, best_val)
        upd_w = wm > worst_val
        worst_idx = jnp.where(upd_w, wi, worst_idx)
        worst_val = jnp.where(upd_w, wm, worst_val)

    bi_ref[...] = best_idx
    wi_ref[...] = worst_idx

    @pl.when(t == 0)
    def _():
        for c in range(_N_CHUNKS):
            cb_raw = et_ref[c * _CODE_CHUNK:(c + 1) * _CODE_CHUNK, :]
            rnorm = jnp.sqrt(jnp.sum(cb_raw * cb_raw, axis=1, keepdims=True))
            cb = (cb_raw / jnp.maximum(rnorm, 1e-12)) * 0.5
            cbn_ref[c * _CODE_CHUNK:(c + 1) * _CODE_CHUNK, :] = (
                jnp.concatenate(
                    [cb, jnp.zeros((_CODE_CHUNK, 128 - _DIM), jnp.float32)],
                    axis=1))

    block_sum = jnp.sum(best_val) + jnp.sum(worst_val)

    @pl.when(t == 0)
    def _():
        acc_ref[0, 0] = 0.0

    acc_ref[0, 0] += block_sum

    @pl.when(t == n_tok_blocks - 1)
    def _():
        diff_ref[...] = jnp.full((1, 1), acc_ref[0, 0] * inv_count,
                                 jnp.float32)


def _tc_stage(x2d, embed, et, n_tok_blocks, inv_count):
    n_tok = x2d.shape[0]
    return pl.pallas_call(
        functools.partial(_tc_body, n_tok_blocks=n_tok_blocks,
                          inv_count=inv_count),
        grid=(n_tok_blocks,),
        in_specs=[
            pl.BlockSpec((_TOK_BLOCK, _DIM), lambda t: (t, 0)),
            pl.BlockSpec((_DIM, _N_EMBED), lambda t: (0, 0)),
            pl.BlockSpec((_N_EMBED, _DIM), lambda t: (0, 0)),
        ],
        out_specs=[
            pl.BlockSpec((_TOK_BLOCK, 1), lambda t: (t, 0)),
            pl.BlockSpec((_TOK_BLOCK, 1), lambda t: (t, 0)),
            pl.BlockSpec((_N_EMBED, 128), lambda t: (0, 0)),
            pl.BlockSpec((1, 1), lambda t: (0, 0)),
        ],
        out_shape=[
            jax.ShapeDtypeStruct((n_tok, 1), jnp.int32),
            jax.ShapeDtypeStruct((n_tok, 1), jnp.int32),
            jax.ShapeDtypeStruct((_N_EMBED, 128), jnp.float32),
            jax.ShapeDtypeStruct((1, 1), jnp.float32),
        ],
        scratch_shapes=[
            pltpu.SMEM((1, 1), jnp.float32),
            pltpu.VMEM((_DIM, _N_EMBED), jnp.float32),
            pltpu.VMEM((1, _N_EMBED), jnp.float32),
        ],
        compiler_params=pltpu.CompilerParams(
            dimension_semantics=("arbitrary",)),
    )(x2d, embed, et)


def _make_sc_gather(n_tok):
    rows_per_w = n_tok // _SC_WORKERS
    n_sub = rows_per_w // 128
    mesh = plsc.VectorSubcoreMesh(core_axis_name="c", subcore_axis_name="s")

    @functools.partial(
        pl.kernel, mesh=mesh,
        out_type=jax.ShapeDtypeStruct((n_tok, 128), jnp.float32),
        scratch_types=[
            pltpu.VMEM((n_sub, 128), jnp.int32),
            pltpu.VMEM((n_sub, 128), jnp.int32),
            pltpu.VMEM((128, 128), jnp.float32),
            pltpu.VMEM((128, 128), jnp.float32),
            pltpu.SemaphoreType.DMA,
            pltpu.SemaphoreType.DMA,
        ],
    )
    def sc_gather(tbl_hbm, bi_hbm, wi_hbm, out_hbm, bi_v, wi_v, q_v, fq_v,
                  sem1, sem2):
        wid = lax.axis_index("s") * _SC_CORES + lax.axis_index("c")
        base = wid * rows_per_w
        pltpu.sync_copy(bi_hbm.at[wid], bi_v)
        pltpu.sync_copy(wi_hbm.at[wid], wi_v)
        for k in range(n_sub):
            cp1 = pltpu.async_copy(tbl_hbm.at[bi_v.at[k]], q_v, sem1)
            cp2 = pltpu.async_copy(tbl_hbm.at[wi_v.at[k]], fq_v, sem2)
            cp1.wait()
            cp2.wait()

            def body(r, carry):
                q_v[r, 0:16] = q_v[r, 0:16] + fq_v[r, 0:16]
                q_v[r, 16:32] = q_v[r, 16:32] + fq_v[r, 16:32]
                return carry

            lax.fori_loop(0, 128, body, 0)
            pltpu.sync_copy(q_v, out_hbm.at[pl.ds(base + k * 128, 128)])

    return sc_gather


def kernel(input, embed):
    n_tok = input.shape[0] * input.shape[1]
    n_tok_blocks = n_tok // _TOK_BLOCK
    inv_count = 1.0 / (n_tok * _DIM)
    x2d = input.reshape(n_tok, _DIM)
    et = embed.T

    bi, wi, cbn_half, diff11 = _tc_stage(x2d, embed, et, n_tok_blocks,
                                         inv_count)
    n_sub = (n_tok // _SC_WORKERS) // 128
    mix_pad = _make_sc_gather(n_tok)(
        cbn_half, bi.reshape(_SC_WORKERS, n_sub, 128),
        wi.reshape(_SC_WORKERS, n_sub, 128))

    mix = mix_pad[:, :_DIM].reshape(input.shape)
    diff = diff11[0, 0]
    return (mix, diff)


# speedup vs baseline: 1.1306x; 1.0103x over previous
import functools

import jax
import jax.numpy as jnp
from jax import lax
from jax.experimental import pallas as pl
from jax.experimental.pallas import tpu as pltpu
from jax.experimental.pallas import tpu_sc as plsc

_DIM = 32
_N_EMBED = 8192
_TOK_BLOCK = 2048
_CODE_CHUNK = 4096
_N_CHUNKS = _N_EMBED // _CODE_CHUNK

_SC_CORES = 2
_SC_SUBCORES = 16
_SC_WORKERS = _SC_CORES * _SC_SUBCORES


def _tc_body(x_ref, e_ref, et_ref, bi_ref, wi_ref, cbn_ref, diff_ref,
             acc_ref, e2_ref, n2_ref, n_tok_blocks, inv_count):
    t = pl.program_id(0)
    x = x_ref[...]
    x2 = jnp.sum(x * x, axis=1, keepdims=True)

    @pl.when(t == 0)
    def _():
        e_raw = e_ref[...]
        norm = jnp.sqrt(jnp.sum(e_raw * e_raw, axis=0, keepdims=True))
        e_n = e_raw / jnp.maximum(norm, 1e-12)
        e2_ref[...] = e_n + e_n
        n2_ref[...] = jnp.sum(e_n * e_n, axis=0, keepdims=True)

    best_val = jnp.full((_TOK_BLOCK, 1), jnp.inf, jnp.float32)
    worst_val = jnp.full((_TOK_BLOCK, 1), -jnp.inf, jnp.float32)
    best_idx = jnp.zeros((_TOK_BLOCK, 1), jnp.int32)
    worst_idx = jnp.zeros((_TOK_BLOCK, 1), jnp.int32)
    iota = jax.lax.broadcasted_iota(
        jnp.int32, (_TOK_BLOCK, _CODE_CHUNK), 1).astype(jnp.float32)

    for c in range(_N_CHUNKS):
        e2 = e2_ref[:, c * _CODE_CHUNK:(c + 1) * _CODE_CHUNK]
        n2 = n2_ref[:, c * _CODE_CHUNK:(c + 1) * _CODE_CHUNK]
        s2 = jax.lax.dot_general(x, e2, (((1,), (0,)), ((), ())),
                                 preferred_element_type=jnp.float32)
        d = (x2 - s2) + n2
        bm = jnp.min(d, axis=1, keepdims=True)
        bi = jnp.min(jnp.where(d == bm, iota, jnp.float32(_N_EMBED)),
                     axis=1, keepdims=True).astype(jnp.int32) + (
                         c * _CODE_CHUNK)
        wm = jnp.max(d, axis=1, keepdims=True)
        wi = jnp.min(jnp.where(d == wm, iota, jnp.float32(_N_EMBED)),
                     axis=1, keepdims=True).astype(jnp.int32) + (
                         c * _CODE_CHUNK)
        upd_b = bm < best_val
        best_idx = jnp.where(upd_b, bi, best_idx)
        best_val = jnp.where(upd_b, bm, best_val)
        upd_w = wm > worst_val
        worst_idx = jnp.where(upd_w, wi, worst_idx)
        worst_val = jnp.where(upd_w, wm, worst_val)

    bi_ref[...] = best_idx
    wi_ref[...] = worst_idx

    @pl.when(t == 0)
    def _():
        for c in range(_N_CHUNKS):
            cb_raw = et_ref[c * _CODE_CHUNK:(c + 1) * _CODE_CHUNK, :]
            rnorm = jnp.sqrt(jnp.sum(cb_raw * cb_raw, axis=1, keepdims=True))
            cb = (cb_raw / jnp.maximum(rnorm, 1e-12)) * 0.5
            cbn_ref[c * _CODE_CHUNK:(c + 1) * _CODE_CHUNK, :] = (
                jnp.concatenate(
                    [cb, jnp.zeros((_CODE_CHUNK, 128 - _DIM), jnp.float32)],
                    axis=1))

    block_sum = jnp.sum(best_val) + jnp.sum(worst_val)

    @pl.when(t == 0)
    def _():
        acc_ref[0, 0] = 0.0

    acc_ref[0, 0] += block_sum

    @pl.when(t == n_tok_blocks - 1)
    def _():
        diff_ref[...] = jnp.full((1, 1), acc_ref[0, 0] * inv_count,
                                 jnp.float32)


def _tc_stage(x2d, embed, et, n_tok_blocks, inv_count):
    n_tok = x2d.shape[0]
    return pl.pallas_call(
        functools.partial(_tc_body, n_tok_blocks=n_tok_blocks,
                          inv_count=inv_count),
        grid=(n_tok_blocks,),
        in_specs=[
            pl.BlockSpec((_TOK_BLOCK, _DIM), lambda t: (t, 0)),
            pl.BlockSpec((_DIM, _N_EMBED), lambda t: (0, 0)),
            pl.BlockSpec((_N_EMBED, _DIM), lambda t: (0, 0)),
        ],
        out_specs=[
            pl.BlockSpec((_TOK_BLOCK, 1), lambda t: (t, 0)),
            pl.BlockSpec((_TOK_BLOCK, 1), lambda t: (t, 0)),
            pl.BlockSpec((_N_EMBED, 128), lambda t: (0, 0)),
            pl.BlockSpec((1, 1), lambda t: (0, 0)),
        ],
        out_shape=[
            jax.ShapeDtypeStruct((n_tok, 1), jnp.int32),
            jax.ShapeDtypeStruct((n_tok, 1), jnp.int32),
            jax.ShapeDtypeStruct((_N_EMBED, 128), jnp.float32),
            jax.ShapeDtypeStruct((1, 1), jnp.float32),
        ],
        scratch_shapes=[
            pltpu.SMEM((1, 1), jnp.float32),
            pltpu.VMEM((_DIM, _N_EMBED), jnp.float32),
            pltpu.VMEM((1, _N_EMBED), jnp.float32),
        ],
        compiler_params=pltpu.CompilerParams(
            dimension_semantics=("arbitrary",)),
    )(x2d, embed, et)


def _make_sc_gather(n_tok):
    rows_per_w = n_tok // _SC_WORKERS
    n_sub = rows_per_w // 128
    mesh = plsc.VectorSubcoreMesh(core_axis_name="c", subcore_axis_name="s")

    @functools.partial(
        pl.kernel, mesh=mesh,
        out_type=jax.ShapeDtypeStruct((n_tok, 128), jnp.float32),
        scratch_types=[
            pltpu.VMEM((n_sub, 128), jnp.int32),
            pltpu.VMEM((n_sub, 128), jnp.int32),
            pltpu.VMEM((128, 128), jnp.float32),
            pltpu.VMEM((128, 128), jnp.float32),
            pltpu.SemaphoreType.DMA,
            pltpu.SemaphoreType.DMA,
        ],
    )
    def sc_gather(tbl_hbm, bi_hbm, wi_hbm, out_hbm, bi_v, wi_v, q_v, fq_v,
                  sem1, sem2):
        wid = lax.axis_index("s") * _SC_CORES + lax.axis_index("c")
        base = wid * rows_per_w
        pltpu.sync_copy(bi_hbm.at[wid], bi_v)
        pltpu.sync_copy(wi_hbm.at[wid], wi_v)
        for k in range(n_sub):
            cp1 = pltpu.async_copy(tbl_hbm.at[bi_v.at[k]], q_v, sem1)
            cp2 = pltpu.async_copy(tbl_hbm.at[wi_v.at[k]], fq_v, sem2)
            cp1.wait()
            cp2.wait()

            def body(r, carry):
                q_v[r, 0:16] = q_v[r, 0:16] + fq_v[r, 0:16]
                q_v[r, 16:32] = q_v[r, 16:32] + fq_v[r, 16:32]
                return carry

            lax.fori_loop(0, 128, body, 0)
            pltpu.sync_copy(q_v, out_hbm.at[pl.ds(base + k * 128, 128)])

    return sc_gather


def kernel(input, embed):
    n_tok = input.shape[0] * input.shape[1]
    n_tok_blocks = n_tok // _TOK_BLOCK
    inv_count = 1.0 / (n_tok * _DIM)
    x2d = input.reshape(n_tok, _DIM)
    et = embed.T

    bi, wi, cbn_half, diff11 = _tc_stage(x2d, embed, et, n_tok_blocks,
                                         inv_count)
    n_sub = (n_tok // _SC_WORKERS) // 128
    mix_pad = _make_sc_gather(n_tok)(
        cbn_half, bi.reshape(_SC_WORKERS, n_sub, 128),
        wi.reshape(_SC_WORKERS, n_sub, 128))

    mix = mix_pad[:, :_DIM].reshape(input.shape)
    diff = diff11[0, 0]
    return (mix, diff)
